# baseline (speedup 1.0000x reference)
import jax
import jax.numpy as jnp
from jax.experimental import pallas as pl
from jax.experimental.pallas import tpu as pltpu

AUG_THRESHOLD = 1024
BIG = 2**30
NBUF_IN = 6
TL = 2048


def _make_outer(B, L, D):
    nt = L // TL

    def outer(lens_ref, ptab_ref, x_hbm, o_hbm):
        def inner(x_ref, o_ref):
            b = pl.program_id(0)
            t = pl.program_id(1)
            ln = lens_ref[b]
            lim = jnp.where(ln > AUG_THRESHOLD, ln, 0)
            mask = ptab_ref[0, pl.ds(t * TL, TL), :] < lim
            o_ref[...] = jnp.where(mask[None], 0.0, x_ref[...])

        pipeline = pltpu.emit_pipeline(
            inner,
            grid=(B, nt),
            in_specs=[
                pl.BlockSpec(
                    (1, TL, D),
                    lambda b, t: (b, t, 0),
                    pipeline_mode=pl.Buffered(
                        buffer_count=NBUF_IN, use_lookahead=True
                    ),
                )
            ],
            out_specs=[pl.BlockSpec((1, TL, D), lambda b, t: (b, t, 0))],
        )
        pipeline(x_hbm, o_hbm)

    return outer


def kernel(sequences, seq_lens):
    B, L, D = sequences.shape
    pos = jnp.arange(L, dtype=jnp.int32)
    ptab = jnp.where(pos % 10 == 0, pos, BIG)[None, :, None]
    out = pl.pallas_call(
        _make_outer(B, L, D),
        grid_spec=pltpu.PrefetchScalarGridSpec(
            num_scalar_prefetch=1,
            grid=(1,),
            in_specs=[
                pl.BlockSpec(memory_space=pltpu.VMEM),
                pl.BlockSpec(memory_space=pltpu.HBM),
            ],
            out_specs=pl.BlockSpec(memory_space=pltpu.HBM),
        ),
        out_shape=jax.ShapeDtypeStruct((B, L, D), sequences.dtype),
    )(seq_lens, ptab, sequences)
    return out, seq_lens


# speedup vs baseline: 1.1355x; 1.1355x over previous
import jax
import jax.numpy as jnp
from jax.experimental import pallas as pl
from jax.experimental.pallas import tpu as pltpu

AUG_THRESHOLD = 1024
BIG16 = 32767
NBUF_IN = 4
BR = 2


def _make_outer(B, L, D):
    def outer(lens_ref, ptab_ref, x_hbm, o_hbm):
        def inner(x_ref, o_ref):
            g = pl.program_id(0)
            ptab = ptab_ref[0]
            for j in range(BR):
                ln = lens_ref[g * BR + j]
                lim = jnp.where(ln > AUG_THRESHOLD, ln, 0).astype(jnp.int16)
                o_ref[j] = jnp.where(ptab < lim, 0.0, x_ref[j])

        pipeline = pltpu.emit_pipeline(
            inner,
            grid=(B // BR,),
            in_specs=[
                pl.BlockSpec(
                    (BR, L, D),
                    lambda g: (g, 0, 0),
                    pipeline_mode=pl.Buffered(
                        buffer_count=NBUF_IN, use_lookahead=True
                    ),
                )
            ],
            out_specs=[pl.BlockSpec((BR, L, D), lambda g: (g, 0, 0))],
        )
        pipeline(x_hbm, o_hbm)

    return outer


def kernel(sequences, seq_lens):
    B, L, D = sequences.shape
    pos = jnp.arange(L, dtype=jnp.int32)
    ptab = jnp.where(pos % 10 == 0, pos, BIG16).astype(jnp.int16)[None, :, None]
    out = pl.pallas_call(
        _make_outer(B, L, D),
        grid_spec=pltpu.PrefetchScalarGridSpec(
            num_scalar_prefetch=1,
            grid=(1,),
            in_specs=[
                pl.BlockSpec(memory_space=pltpu.VMEM),
                pl.BlockSpec(memory_space=pltpu.HBM),
            ],
            out_specs=pl.BlockSpec(memory_space=pltpu.HBM),
        ),
        out_shape=jax.ShapeDtypeStruct((B, L, D), sequences.dtype),
    )(seq_lens, ptab, sequences)
    return out, seq_lens


# speedup vs baseline: 1.1589x; 1.0206x over previous
import jax
import jax.numpy as jnp
from jax.experimental import pallas as pl
from jax.experimental.pallas import tpu as pltpu

AUG_THRESHOLD = 1024
BIG16 = 32767
NBUF_IN = 4
BR = 4


def _make_outer(B, L, D):
    def outer(lens_ref, ptab_ref, x_hbm, o_hbm):
        def inner(x_ref, o_ref):
            g = pl.program_id(0)
            ptab = ptab_ref[0]
            for j in range(BR):
                ln = lens_ref[g * BR + j]
                lim = jnp.where(ln > AUG_THRESHOLD, ln, 0).astype(jnp.int16)
                o_ref[j] = jnp.where(ptab < lim, 0.0, x_ref[j])

        pipeline = pltpu.emit_pipeline(
            inner,
            grid=(B // BR,),
            in_specs=[
                pl.BlockSpec(
                    (BR, L, D),
                    lambda g: (g, 0, 0),
                    pipeline_mode=pl.Buffered(
                        buffer_count=NBUF_IN, use_lookahead=True
                    ),
                )
            ],
            out_specs=[pl.BlockSpec((BR, L, D), lambda g: (g, 0, 0))],
        )
        pipeline(x_hbm, o_hbm)

    return outer


def kernel(sequences, seq_lens):
    B, L, D = sequences.shape
    pos = jnp.arange(L, dtype=jnp.int32)
    ptab = jnp.where(pos % 10 == 0, pos, BIG16).astype(jnp.int16)[None, :, None]
    out = pl.pallas_call(
        _make_outer(B, L, D),
        grid_spec=pltpu.PrefetchScalarGridSpec(
            num_scalar_prefetch=1,
            grid=(1,),
            in_specs=[
                pl.BlockSpec(memory_space=pltpu.VMEM),
                pl.BlockSpec(memory_space=pltpu.HBM),
            ],
            out_specs=pl.BlockSpec(memory_space=pltpu.HBM),
        ),
        out_shape=jax.ShapeDtypeStruct((B, L, D), sequences.dtype),
    )(seq_lens, ptab, sequences)
    return out, seq_lens
